# Initial kernel scaffold; baseline (speedup 1.0000x reference)
import jax
import jax.numpy as jnp
from jax.experimental import pallas as pl


def kernel(hidden_states, gate_weight, gate_proj_w, up_proj_w, down_proj_w):
    raise NotImplementedError("write your pallas kernel here")


# speedup vs baseline: 9.7854x; 9.7854x over previous
import functools

import jax
import jax.numpy as jnp
from jax import lax
from jax.experimental import pallas as pl
from jax.experimental.pallas import tpu as pltpu
from jax.experimental.pallas import tpu_sc as plsc

E = 64
TOPK = 2
DIM = 1024
FFN = 512
SEQ = 2048
N = SEQ * TOPK
T = 128
PADN = N + E * T
NT = PADN // T

NC = 2
NS = 16
NW = NC * NS
CHUNK = N // NW
SUB = 64


def _route_body(x_ref, gw_ref, pos_ref, w_ref, eid_ref, nact_ref):
    x = x_ref[...]
    gw = gw_ref[...]
    logits = lax.dot_general(x, gw, (((1,), (1,)), ((), ())),
                             preferred_element_type=jnp.float32)
    m = jnp.max(logits, axis=1, keepdims=True)
    ex = jnp.exp(logits - m)
    scores = ex / jnp.sum(ex, axis=1, keepdims=True)

    eidx = lax.broadcasted_iota(jnp.int32, (SEQ, E), 1)
    m1 = jnp.max(scores, axis=1, keepdims=True)
    a1 = jnp.min(jnp.where(scores == m1, eidx, E), axis=1, keepdims=True)
    s2 = jnp.where(eidx == a1, -jnp.inf, scores)
    m2 = jnp.max(s2, axis=1, keepdims=True)
    a2 = jnp.min(jnp.where(s2 == m2, eidx, E), axis=1, keepdims=True)

    denom = m1 + m2 + 1e-20
    w_ref[:, 0:1] = m1 / denom
    w_ref[:, 1:2] = m2 / denom

    one0 = (eidx == a1).astype(jnp.float32)
    one1 = (eidx == a2).astype(jnp.float32)
    both = one0 + one1

    counts = jnp.sum(both, axis=0, keepdims=True)
    pc = ((counts.astype(jnp.int32) + (T - 1)) // T) * T
    pcf = pc.astype(jnp.float32)
    er = lax.broadcasted_iota(jnp.int32, (E, E), 0)
    ec = lax.broadcasted_iota(jnp.int32, (E, E), 1)
    upper = (er < ec).astype(jnp.float32)
    poff = lax.dot_general(pcf, upper, (((1,), (0,)), ((), ())),
                           preferred_element_type=jnp.float32)
    total = jnp.sum(pcf, axis=1, keepdims=True)

    B = 512
    br = lax.broadcasted_iota(jnp.int32, (B, B), 0)
    bc = lax.broadcasted_iota(jnp.int32, (B, B), 1)
    ltri = (bc < br).astype(jnp.float32)
    carry = jnp.zeros((1, E), jnp.float32)
    for b in range(SEQ // B):
        sl = slice(b * B, (b + 1) * B)
        cb = both[sl]
        run = lax.dot_general(ltri, cb, (((1,), (0,)), ((), ())),
                              preferred_element_type=jnp.float32) + carry
        carry = carry + jnp.sum(cb, axis=0, keepdims=True)
        dest = run + poff
        pos_ref[sl, 0:1] = jnp.sum(one0[sl] * dest, axis=1,
                                   keepdims=True).astype(jnp.int32)
        pos_ref[sl, 1:2] = jnp.sum(one1[sl] * dest, axis=1,
                                   keepdims=True).astype(jnp.int32)

    tstart = (lax.broadcasted_iota(jnp.int32, (NT, 1), 0) * T).astype(jnp.float32)
    p = jnp.minimum(tstart, total - 1.0)
    eid_ref[...] = jnp.sum((poff <= p).astype(jnp.int32), axis=1,
                           keepdims=True) - 1
    nact_ref[...] = (total.astype(jnp.int32) // T)


_route = pl.pallas_call(
    _route_body,
    out_shape=(
        jax.ShapeDtypeStruct((SEQ, TOPK), jnp.int32),
        jax.ShapeDtypeStruct((SEQ, TOPK), jnp.float32),
        jax.ShapeDtypeStruct((NT, 1), jnp.int32),
        jax.ShapeDtypeStruct((1, 1), jnp.int32),
    ),
)


@functools.lru_cache(maxsize=None)
def _sc_kernels():
    mesh = plsc.VectorSubcoreMesh(core_axis_name="c", subcore_axis_name="s")

    @functools.partial(
        pl.kernel,
        out_type=jax.ShapeDtypeStruct((PADN, DIM), jnp.float32),
        mesh=mesh,
        scratch_types=[
            pltpu.VMEM((SUB,), jnp.int32),
            pltpu.VMEM((SUB,), jnp.int32),
            pltpu.VMEM((SUB, DIM), jnp.float32),
            pltpu.SemaphoreType.DMA,
        ],
    )
    def _dispatch(x_hbm, pos_hbm, src_hbm, out_hbm, src_v, dst_v, rows_v, sem):
        wid = lax.axis_index("s") * NC + lax.axis_index("c")
        for c in range(CHUNK // SUB):
            base = wid * CHUNK + c * SUB
            pltpu.sync_copy(src_hbm.at[pl.ds(base, SUB)], src_v)
            pltpu.sync_copy(pos_hbm.at[pl.ds(base, SUB)], dst_v)
            pltpu.async_copy(x_hbm.at[src_v], rows_v, sem).wait()
            pltpu.async_copy(rows_v, out_hbm.at[dst_v], sem).wait()

    @functools.partial(
        pl.kernel,
        out_type=jax.ShapeDtypeStruct((N, DIM), jnp.float32),
        mesh=mesh,
        scratch_types=[
            pltpu.VMEM((SUB,), jnp.int32),
            pltpu.VMEM((SUB, DIM), jnp.float32),
            pltpu.SemaphoreType.DMA,
        ],
    )
    def _unsort(ffn_hbm, pos_hbm, out_hbm, idx_v, rows_v, sem):
        wid = lax.axis_index("s") * NC + lax.axis_index("c")
        for c in range(CHUNK // SUB):
            base = wid * CHUNK + c * SUB
            pltpu.sync_copy(pos_hbm.at[pl.ds(base, SUB)], idx_v)
            pltpu.async_copy(ffn_hbm.at[idx_v], rows_v, sem).wait()
            pltpu.sync_copy(rows_v, out_hbm.at[pl.ds(base, SUB)])

    return _dispatch, _unsort


def _gemm_body(eid_ref, nact_ref, x_ref, wg_ref, wu_ref, wd_ref, o_ref):
    t = pl.program_id(0)

    @pl.when(t < nact_ref[0])
    def _():
        xt = x_ref[...]
        g = lax.dot_general(xt, wg_ref[0], (((1,), (1,)), ((), ())),
                            preferred_element_type=jnp.float32)
        u = lax.dot_general(xt, wu_ref[0], (((1,), (1,)), ((), ())),
                            preferred_element_type=jnp.float32)
        h = g * (1.0 / (1.0 + jnp.exp(-g))) * u
        o_ref[...] = lax.dot_general(h, wd_ref[0], (((1,), (1,)), ((), ())),
                                     preferred_element_type=jnp.float32)


_gemm = pl.pallas_call(
    _gemm_body,
    grid_spec=pltpu.PrefetchScalarGridSpec(
        num_scalar_prefetch=2,
        grid=(NT,),
        in_specs=[
            pl.BlockSpec((T, DIM), lambda t, eid, na: (t, 0)),
            pl.BlockSpec((1, FFN, DIM), lambda t, eid, na: (eid[t], 0, 0)),
            pl.BlockSpec((1, FFN, DIM), lambda t, eid, na: (eid[t], 0, 0)),
            pl.BlockSpec((1, DIM, FFN), lambda t, eid, na: (eid[t], 0, 0)),
        ],
        out_specs=pl.BlockSpec((T, DIM), lambda t, eid, na: (t, 0)),
    ),
    out_shape=jax.ShapeDtypeStruct((PADN, DIM), jnp.float32),
    compiler_params=pltpu.CompilerParams(
        dimension_semantics=("arbitrary",)),
)


RB = 256


def _combine_body(u_ref, w_ref, o_ref):
    u = u_ref[...]
    w = w_ref[...]
    o_ref[...] = u[:, :DIM] * w[:, 0:1] + u[:, DIM:] * w[:, 1:2]


_combine = pl.pallas_call(
    _combine_body,
    grid=(SEQ // RB,),
    in_specs=[
        pl.BlockSpec((RB, TOPK * DIM), lambda i: (i, 0)),
        pl.BlockSpec((RB, TOPK), lambda i: (i, 0)),
    ],
    out_specs=pl.BlockSpec((RB, DIM), lambda i: (i, 0)),
    out_shape=jax.ShapeDtypeStruct((SEQ, DIM), jnp.float32),
)


def kernel(hidden_states, gate_weight, gate_proj_w, up_proj_w, down_proj_w):
    b, s, h = hidden_states.shape
    x = hidden_states.reshape(SEQ, DIM).astype(jnp.float32)
    pos2, w2, eid2, nact2 = _route(x, gate_weight)
    pos = pos2.reshape(N)
    src = jnp.arange(N, dtype=jnp.int32) // TOPK
    _dispatch, _unsort = _sc_kernels()
    sorted_x = _dispatch(x, pos, src)
    ffn = _gemm(eid2.reshape(NT), nact2.reshape(1), sorted_x,
                gate_proj_w, up_proj_w, down_proj_w)
    unsorted = _unsort(ffn, pos)
    out = _combine(unsorted.reshape(SEQ, TOPK * DIM), w2)
    return out.reshape(b, s, h)


# speedup vs baseline: 11.6651x; 1.1921x over previous
import functools

import jax
import jax.numpy as jnp
from jax import lax
from jax.experimental import pallas as pl
from jax.experimental.pallas import tpu as pltpu
from jax.experimental.pallas import tpu_sc as plsc

E = 64
TOPK = 2
DIM = 1024
FFN = 512
SEQ = 2048
N = SEQ * TOPK
T = 128
PADN = N + E * T
NT = PADN // T

NC = 2
NS = 16
NW = NC * NS
TPW = SEQ // NW
CT = 32
LW = 16


def _route_body(x_ref, gw_ref, pos0_ref, pos1_ref, w_ref, eid_ref, nact_ref):
    x = x_ref[...]
    gw = gw_ref[...]
    logits = lax.dot_general(x, gw, (((1,), (1,)), ((), ())),
                             preferred_element_type=jnp.float32)
    m = jnp.max(logits, axis=1, keepdims=True)
    ex = jnp.exp(logits - m)
    scores = ex / jnp.sum(ex, axis=1, keepdims=True)

    eidx = lax.broadcasted_iota(jnp.int32, (SEQ, E), 1)
    m1 = jnp.max(scores, axis=1, keepdims=True)
    a1 = jnp.min(jnp.where(scores == m1, eidx, E), axis=1, keepdims=True)
    s2 = jnp.where(eidx == a1, -jnp.inf, scores)
    m2 = jnp.max(s2, axis=1, keepdims=True)
    a2 = jnp.min(jnp.where(s2 == m2, eidx, E), axis=1, keepdims=True)

    denom = m1 + m2 + 1e-20
    w_ref[:, :LW] = jnp.broadcast_to(m1 / denom, (SEQ, LW))
    w_ref[:, LW:] = jnp.broadcast_to(m2 / denom, (SEQ, LW))

    one0 = (eidx == a1).astype(jnp.float32)
    one1 = (eidx == a2).astype(jnp.float32)
    both = one0 + one1

    counts = jnp.sum(both, axis=0, keepdims=True)
    pc = ((counts.astype(jnp.int32) + (T - 1)) // T) * T
    pcf = pc.astype(jnp.float32)
    er = lax.broadcasted_iota(jnp.int32, (E, E), 0)
    ec = lax.broadcasted_iota(jnp.int32, (E, E), 1)
    upper = (er < ec).astype(jnp.float32)
    poff = lax.dot_general(pcf, upper, (((1,), (0,)), ((), ())),
                           preferred_element_type=jnp.float32)
    total = jnp.sum(pcf, axis=1, keepdims=True)

    B = 512
    br = lax.broadcasted_iota(jnp.int32, (B, B), 0)
    bc = lax.broadcasted_iota(jnp.int32, (B, B), 1)
    ltri = (bc < br).astype(jnp.float32)
    carry = jnp.zeros((1, E), jnp.float32)
    for b in range(SEQ // B):
        sl = slice(b * B, (b + 1) * B)
        cb = both[sl]
        run = lax.dot_general(ltri, cb, (((1,), (0,)), ((), ())),
                              preferred_element_type=jnp.float32) + carry
        carry = carry + jnp.sum(cb, axis=0, keepdims=True)
        dest = run + poff
        pos0_ref[sl, :] = jnp.sum(one0[sl] * dest, axis=1,
                                  keepdims=True).astype(jnp.int32)
        pos1_ref[sl, :] = jnp.sum(one1[sl] * dest, axis=1,
                                  keepdims=True).astype(jnp.int32)

    tstart = (lax.broadcasted_iota(jnp.int32, (NT, 1), 0) * T).astype(jnp.float32)
    p = jnp.minimum(tstart, total - 1.0)
    eid_ref[...] = jnp.sum((poff <= p).astype(jnp.int32), axis=1,
                           keepdims=True) - 1
    nact_ref[...] = (total.astype(jnp.int32) // T)


_route = pl.pallas_call(
    _route_body,
    out_shape=(
        jax.ShapeDtypeStruct((SEQ, 1), jnp.int32),
        jax.ShapeDtypeStruct((SEQ, 1), jnp.int32),
        jax.ShapeDtypeStruct((SEQ, TOPK * LW), jnp.float32),
        jax.ShapeDtypeStruct((NT, 1), jnp.int32),
        jax.ShapeDtypeStruct((1, 1), jnp.int32),
    ),
)


@functools.lru_cache(maxsize=None)
def _sc_kernels():
    mesh = plsc.VectorSubcoreMesh(core_axis_name="c", subcore_axis_name="s")

    @functools.partial(
        pl.kernel,
        out_type=jax.ShapeDtypeStruct((PADN, DIM), jnp.float32),
        mesh=mesh,
        scratch_types=[
            pltpu.VMEM((TPW,), jnp.int32),
            pltpu.VMEM((TPW,), jnp.int32),
            pltpu.VMEM((TPW, DIM), jnp.float32),
            pltpu.SemaphoreType.DMA,
        ],
    )
    def _dispatch(x_hbm, pos0_hbm, pos1_hbm, out_hbm, d0_v, d1_v, rows_v, sem):
        wid = lax.axis_index("s") * NC + lax.axis_index("c")
        base = wid * TPW
        pltpu.sync_copy(pos0_hbm.at[pl.ds(base, TPW)], d0_v)
        pltpu.sync_copy(pos1_hbm.at[pl.ds(base, TPW)], d1_v)
        pltpu.sync_copy(x_hbm.at[pl.ds(base, TPW)], rows_v)
        c0 = pltpu.async_copy(rows_v, out_hbm.at[d0_v], sem)
        c1 = pltpu.async_copy(rows_v, out_hbm.at[d1_v], sem)
        c0.wait()
        c1.wait()

    @functools.partial(
        pl.kernel,
        out_type=jax.ShapeDtypeStruct((SEQ, DIM), jnp.float32),
        mesh=mesh,
        scratch_types=[
            pltpu.VMEM((CT,), jnp.int32),
            pltpu.VMEM((CT,), jnp.int32),
            pltpu.VMEM((CT, LW), jnp.float32),
            pltpu.VMEM((CT, LW), jnp.float32),
            pltpu.VMEM((CT, DIM), jnp.float32),
            pltpu.VMEM((CT, DIM), jnp.float32),
            pltpu.VMEM((CT, DIM), jnp.float32),
            pltpu.SemaphoreType.DMA,
            pltpu.SemaphoreType.DMA,
        ],
    )
    def _comb(ffn_hbm, pos0_hbm, pos1_hbm, w0_hbm, w1_hbm, out_hbm,
              i0_v, i1_v, w0_v, w1_v, ra_v, rb_v, ro_v, sema, semb):
        wid = lax.axis_index("s") * NC + lax.axis_index("c")
        for c in range(TPW // CT):
            base = wid * TPW + c * CT
            pltpu.sync_copy(pos0_hbm.at[pl.ds(base, CT)], i0_v)
            pltpu.sync_copy(pos1_hbm.at[pl.ds(base, CT)], i1_v)
            pltpu.sync_copy(w0_hbm.at[pl.ds(base, CT)], w0_v)
            pltpu.sync_copy(w1_hbm.at[pl.ds(base, CT)], w1_v)
            ca = pltpu.async_copy(ffn_hbm.at[i0_v], ra_v, sema)
            cb = pltpu.async_copy(ffn_hbm.at[i1_v], rb_v, semb)
            ca.wait()
            cb.wait()

            def tok(j, carry):
                wa = w0_v[j]
                wb = w1_v[j]
                for k in range(DIM // LW):
                    sl = pl.ds(k * LW, LW)
                    ro_v[j, sl] = wa * ra_v[j, sl] + wb * rb_v[j, sl]
                return carry

            lax.fori_loop(0, CT, tok, 0)
            pltpu.sync_copy(ro_v, out_hbm.at[pl.ds(base, CT)])

    return _dispatch, _comb


def _gemm_body(eid_ref, nact_ref, x_ref, wg_ref, wu_ref, wd_ref, o_ref):
    t = pl.program_id(0)

    @pl.when(t < nact_ref[0])
    def _():
        xt = x_ref[...]
        g = lax.dot_general(xt, wg_ref[0], (((1,), (1,)), ((), ())),
                            preferred_element_type=jnp.float32)
        u = lax.dot_general(xt, wu_ref[0], (((1,), (1,)), ((), ())),
                            preferred_element_type=jnp.float32)
        h = g * (1.0 / (1.0 + jnp.exp(-g))) * u
        o_ref[...] = lax.dot_general(h, wd_ref[0], (((1,), (1,)), ((), ())),
                                     preferred_element_type=jnp.float32)


_gemm = pl.pallas_call(
    _gemm_body,
    grid_spec=pltpu.PrefetchScalarGridSpec(
        num_scalar_prefetch=2,
        grid=(NT,),
        in_specs=[
            pl.BlockSpec((T, DIM),
                         lambda t, eid, na: (jnp.minimum(t, na[0] - 1), 0)),
            pl.BlockSpec((1, FFN, DIM), lambda t, eid, na: (eid[t], 0, 0)),
            pl.BlockSpec((1, FFN, DIM), lambda t, eid, na: (eid[t], 0, 0)),
            pl.BlockSpec((1, DIM, FFN), lambda t, eid, na: (eid[t], 0, 0)),
        ],
        out_specs=pl.BlockSpec(
            (T, DIM), lambda t, eid, na: (jnp.where(t < na[0], t, NT - 1), 0)),
    ),
    out_shape=jax.ShapeDtypeStruct((PADN, DIM), jnp.float32),
    compiler_params=pltpu.CompilerParams(
        dimension_semantics=("arbitrary",)),
)


def kernel(hidden_states, gate_weight, gate_proj_w, up_proj_w, down_proj_w):
    b, s, h = hidden_states.shape
    x = hidden_states.reshape(SEQ, DIM).astype(jnp.float32)
    pos0, pos1, wexp, eid2, nact2 = _route(x, gate_weight)
    pos0 = pos0.reshape(SEQ)
    pos1 = pos1.reshape(SEQ)
    _dispatch, _comb = _sc_kernels()
    sorted_x = _dispatch(x, pos0, pos1)
    ffn = _gemm(eid2.reshape(NT), nact2.reshape(1), sorted_x,
                gate_proj_w, up_proj_w, down_proj_w)
    out = _comb(ffn, pos0, pos1, wexp[:, :LW], wexp[:, LW:])
    return out.reshape(b, s, h)


# speedup vs baseline: 11.7619x; 1.0083x over previous
import functools

import jax
import jax.numpy as jnp
from jax import lax
from jax.experimental import pallas as pl
from jax.experimental.pallas import tpu as pltpu
from jax.experimental.pallas import tpu_sc as plsc

E = 64
TOPK = 2
DIM = 1024
FFN = 512
SEQ = 2048
N = SEQ * TOPK
T = 128
PADN = N + E * T
NT = PADN // T

NC = 2
NS = 16
NW = NC * NS
TPW = SEQ // NW
CT = 32
LW = 16


def _route_body(x_ref, gw_ref, pos0_ref, pos1_ref, w_ref, eid_ref, nact_ref):
    x = x_ref[...]
    gw = gw_ref[...]
    logits = lax.dot_general(x, gw, (((1,), (1,)), ((), ())),
                             preferred_element_type=jnp.float32)
    m = jnp.max(logits, axis=1, keepdims=True)
    ex = jnp.exp(logits - m)
    scores = ex / jnp.sum(ex, axis=1, keepdims=True)

    eidx = lax.broadcasted_iota(jnp.int32, (SEQ, E), 1)
    m1 = jnp.max(scores, axis=1, keepdims=True)
    a1 = jnp.min(jnp.where(scores == m1, eidx, E), axis=1, keepdims=True)
    s2 = jnp.where(eidx == a1, -jnp.inf, scores)
    m2 = jnp.max(s2, axis=1, keepdims=True)
    a2 = jnp.min(jnp.where(s2 == m2, eidx, E), axis=1, keepdims=True)

    denom = m1 + m2 + 1e-20
    w_ref[:, :LW] = jnp.broadcast_to(m1 / denom, (SEQ, LW))
    w_ref[:, LW:] = jnp.broadcast_to(m2 / denom, (SEQ, LW))

    one0 = (eidx == a1).astype(jnp.float32)
    one1 = (eidx == a2).astype(jnp.float32)
    both = one0 + one1

    counts = jnp.sum(both, axis=0, keepdims=True)
    pc = ((counts.astype(jnp.int32) + (T - 1)) // T) * T
    pcf = pc.astype(jnp.float32)
    er = lax.broadcasted_iota(jnp.int32, (E, E), 0)
    ec = lax.broadcasted_iota(jnp.int32, (E, E), 1)
    upper = (er < ec).astype(jnp.float32)
    poff = lax.dot_general(pcf, upper, (((1,), (0,)), ((), ())),
                           preferred_element_type=jnp.float32)
    total = jnp.sum(pcf, axis=1, keepdims=True)

    B = 512
    br = lax.broadcasted_iota(jnp.int32, (B, B), 0)
    bc = lax.broadcasted_iota(jnp.int32, (B, B), 1)
    ltri = (bc < br).astype(jnp.float32)
    carry = jnp.zeros((1, E), jnp.float32)
    for b in range(SEQ // B):
        sl = slice(b * B, (b + 1) * B)
        cb = both[sl]
        run = lax.dot_general(ltri, cb, (((1,), (0,)), ((), ())),
                              preferred_element_type=jnp.float32) + carry
        carry = carry + jnp.sum(cb, axis=0, keepdims=True)
        dest = run + poff
        pos0_ref[sl, :] = jnp.sum(one0[sl] * dest, axis=1,
                                  keepdims=True).astype(jnp.int32)
        pos1_ref[sl, :] = jnp.sum(one1[sl] * dest, axis=1,
                                  keepdims=True).astype(jnp.int32)

    tstart = (lax.broadcasted_iota(jnp.int32, (NT, 1), 0) * T).astype(jnp.float32)
    p = jnp.minimum(tstart, total - 1.0)
    eid_ref[...] = jnp.sum((poff <= p).astype(jnp.int32), axis=1,
                           keepdims=True) - 1
    nact_ref[...] = (total.astype(jnp.int32) // T)


_route = pl.pallas_call(
    _route_body,
    out_shape=(
        jax.ShapeDtypeStruct((SEQ, 1), jnp.int32),
        jax.ShapeDtypeStruct((SEQ, 1), jnp.int32),
        jax.ShapeDtypeStruct((SEQ, TOPK * LW), jnp.float32),
        jax.ShapeDtypeStruct((NT, 1), jnp.int32),
        jax.ShapeDtypeStruct((1, 1), jnp.int32),
    ),
)


@functools.lru_cache(maxsize=None)
def _sc_kernels():
    mesh = plsc.VectorSubcoreMesh(core_axis_name="c", subcore_axis_name="s")

    @functools.partial(
        pl.kernel,
        out_type=jax.ShapeDtypeStruct((PADN, DIM), jnp.float32),
        mesh=mesh,
        scratch_types=[
            pltpu.VMEM((TPW,), jnp.int32),
            pltpu.VMEM((TPW,), jnp.int32),
            pltpu.VMEM((TPW, DIM), jnp.float32),
            pltpu.SemaphoreType.DMA,
        ],
    )
    def _dispatch(x_hbm, pos0_hbm, pos1_hbm, out_hbm, d0_v, d1_v, rows_v, sem):
        wid = lax.axis_index("s") * NC + lax.axis_index("c")
        base = wid * TPW
        pltpu.sync_copy(pos0_hbm.at[pl.ds(base, TPW)], d0_v)
        pltpu.sync_copy(pos1_hbm.at[pl.ds(base, TPW)], d1_v)
        pltpu.sync_copy(x_hbm.at[pl.ds(base, TPW)], rows_v)
        c0 = pltpu.async_copy(rows_v, out_hbm.at[d0_v], sem)
        c1 = pltpu.async_copy(rows_v, out_hbm.at[d1_v], sem)
        c0.wait()
        c1.wait()

    @functools.partial(
        pl.kernel,
        out_type=jax.ShapeDtypeStruct((SEQ, DIM), jnp.float32),
        mesh=mesh,
        scratch_types=[
            pltpu.VMEM((CT,), jnp.int32),
            pltpu.VMEM((CT,), jnp.int32),
            pltpu.VMEM((CT, LW), jnp.float32),
            pltpu.VMEM((CT, LW), jnp.float32),
            pltpu.VMEM((CT, DIM), jnp.float32),
            pltpu.VMEM((CT, DIM), jnp.float32),
            pltpu.VMEM((CT, DIM), jnp.float32),
            pltpu.SemaphoreType.DMA,
            pltpu.SemaphoreType.DMA,
        ],
    )
    def _comb(ffn_hbm, pos0_hbm, pos1_hbm, w0_hbm, w1_hbm, out_hbm,
              i0_v, i1_v, w0_v, w1_v, ra_v, rb_v, ro_v, sema, semb):
        wid = lax.axis_index("s") * NC + lax.axis_index("c")
        for c in range(TPW // CT):
            base = wid * TPW + c * CT
            pltpu.sync_copy(pos0_hbm.at[pl.ds(base, CT)], i0_v)
            pltpu.sync_copy(pos1_hbm.at[pl.ds(base, CT)], i1_v)
            pltpu.sync_copy(w0_hbm.at[pl.ds(base, CT)], w0_v)
            pltpu.sync_copy(w1_hbm.at[pl.ds(base, CT)], w1_v)
            ca = pltpu.async_copy(ffn_hbm.at[i0_v], ra_v, sema)
            cb = pltpu.async_copy(ffn_hbm.at[i1_v], rb_v, semb)
            ca.wait()
            cb.wait()

            @plsc.parallel_loop(0, CT)
            def _tok(j):
                wa = w0_v[j]
                wb = w1_v[j]
                for k in range(DIM // LW):
                    sl = pl.ds(k * LW, LW)
                    ro_v[j, sl] = wa * ra_v[j, sl] + wb * rb_v[j, sl]

            pltpu.sync_copy(ro_v, out_hbm.at[pl.ds(base, CT)])

    return _dispatch, _comb


def _gemm_body(eid_ref, nact_ref, x_ref, wg_ref, wu_ref, wd_ref, o_ref):
    t = pl.program_id(0)

    @pl.when(t < nact_ref[0])
    def _():
        xt = x_ref[...]
        g = lax.dot_general(xt, wg_ref[0], (((1,), (1,)), ((), ())),
                            preferred_element_type=jnp.float32)
        u = lax.dot_general(xt, wu_ref[0], (((1,), (1,)), ((), ())),
                            preferred_element_type=jnp.float32)
        h = g * (1.0 / (1.0 + jnp.exp(-g))) * u
        o_ref[...] = lax.dot_general(h, wd_ref[0], (((1,), (1,)), ((), ())),
                                     preferred_element_type=jnp.float32)


_gemm = pl.pallas_call(
    _gemm_body,
    grid_spec=pltpu.PrefetchScalarGridSpec(
        num_scalar_prefetch=2,
        grid=(NT,),
        in_specs=[
            pl.BlockSpec((T, DIM),
                         lambda t, eid, na: (jnp.minimum(t, na[0] - 1), 0)),
            pl.BlockSpec((1, FFN, DIM), lambda t, eid, na: (eid[t], 0, 0)),
            pl.BlockSpec((1, FFN, DIM), lambda t, eid, na: (eid[t], 0, 0)),
            pl.BlockSpec((1, DIM, FFN), lambda t, eid, na: (eid[t], 0, 0)),
        ],
        out_specs=pl.BlockSpec(
            (T, DIM), lambda t, eid, na: (jnp.where(t < na[0], t, NT - 1), 0)),
    ),
    out_shape=jax.ShapeDtypeStruct((PADN, DIM), jnp.float32),
    compiler_params=pltpu.CompilerParams(
        dimension_semantics=("arbitrary",)),
)


def kernel(hidden_states, gate_weight, gate_proj_w, up_proj_w, down_proj_w):
    b, s, h = hidden_states.shape
    x = hidden_states.reshape(SEQ, DIM).astype(jnp.float32)
    pos0, pos1, wexp, eid2, nact2 = _route(x, gate_weight)
    pos0 = pos0.reshape(SEQ)
    pos1 = pos1.reshape(SEQ)
    _dispatch, _comb = _sc_kernels()
    sorted_x = _dispatch(x, pos0, pos1)
    ffn = _gemm(eid2.reshape(NT), nact2.reshape(1), sorted_x,
                gate_proj_w, up_proj_w, down_proj_w)
    out = _comb(ffn, pos0, pos1, wexp[:, :LW], wexp[:, LW:])
    return out.reshape(b, s, h)
